# Initial kernel scaffold; baseline (speedup 1.0000x reference)
import jax
import jax.numpy as jnp
from jax.experimental import pallas as pl


def kernel(nf, edge_index, ef, W_e1, W_e2, W_n1, W_n2):
    raise NotImplementedError("write your pallas kernel here")


# speedup vs baseline: 1.0197x; 1.0197x over previous
import jax
import jax.numpy as jnp
from jax.experimental import pallas as pl

N = 10000
E = 320000
D = 128

_ROW_BLK = 400


def _node_mlp_body(nf_ref, red_ref, wn1a_ref, wn1b_ref, wn2_ref, out_ref):
    x = nf_ref[...] @ wn1a_ref[...] + red_ref[...] @ wn1b_ref[...]
    h = jnp.maximum(x, 0.01 * x)
    out_ref[...] = h @ wn2_ref[...]


def _node_mlp(nf, red, W_n1, W_n2):
    wn1a = W_n1[:D]
    wn1b = W_n1[D:]
    grid = (N // _ROW_BLK,)
    return pl.pallas_call(
        _node_mlp_body,
        grid=grid,
        in_specs=[
            pl.BlockSpec((_ROW_BLK, D), lambda i: (i, 0)),
            pl.BlockSpec((_ROW_BLK, D), lambda i: (i, 0)),
            pl.BlockSpec((D, D), lambda i: (0, 0)),
            pl.BlockSpec((D, D), lambda i: (0, 0)),
            pl.BlockSpec((D, D), lambda i: (0, 0)),
        ],
        out_specs=pl.BlockSpec((_ROW_BLK, D), lambda i: (i, 0)),
        out_shape=jax.ShapeDtypeStruct((N, D), jnp.float32),
    )(nf, red, wn1a, wn1b, wn2_ := W_n2)


def kernel(nf, edge_index, ef, W_e1, W_e2, W_n1, W_n2):
    src = edge_index[0]
    dst = edge_index[1]
    A = nf @ W_e1[:D]
    B = nf @ W_e1[D:2 * D]
    C = ef @ W_e1[2 * D:]
    src_full = jnp.concatenate([src, dst])
    dst_full = jnp.concatenate([dst, src])
    C_full = jnp.concatenate([C, C])
    pre = jnp.take(A, src_full, axis=0) + jnp.take(B, dst_full, axis=0) + C_full
    h = jnp.maximum(pre, 0.01 * pre)
    S = jax.ops.segment_sum(h, dst_full, num_segments=N)
    red = S @ W_e2
    return _node_mlp(nf, red, W_n1, W_n2)


# speedup vs baseline: 5.6880x; 5.5782x over previous
import functools

import jax
import jax.numpy as jnp
from jax import lax
from jax.experimental import pallas as pl
from jax.experimental.pallas import tpu as pltpu
from jax.experimental.pallas import tpu_sc as plsc

N = 10000
E = 320000
D = 128

NC = 2
NS = 16
NW = NC * NS

SPAD = 10240
ROWS_PER_SUB = SPAD // NS
EDGES_PER_SUB = E // NW
K = 40
NCHUNK = EDGES_PER_SUB // K


def _edge_body(a_hbm, b_hbm, c_hbm, src_hbm, dst_hbm, out_hbm,
               s_sh, idx_s, idx_d, r_as, r_bd, r_ad, r_bs, r_c, sem):
    cid = lax.axis_index("c")
    sid = lax.axis_index("s")
    wid = cid * NS + sid

    def _zrow(i, _):
        for j in range(D // 16):
            r_c[i, pl.ds(j * 16, 16)] = jnp.zeros((16,), jnp.float32)
        return _
    lax.fori_loop(0, K, _zrow, None)

    def _zchunk(t, _):
        pltpu.sync_copy(r_c, s_sh.at[pl.ds(sid * ROWS_PER_SUB + t * K, K)])
        return _
    lax.fori_loop(0, ROWS_PER_SUB // K, _zchunk, None)
    plsc.subcore_barrier()

    ebase = wid * EDGES_PER_SUB

    def _chunk(t, _):
        base = ebase + t * K
        pltpu.sync_copy(src_hbm.at[pl.ds(base, K)], idx_s)
        pltpu.sync_copy(dst_hbm.at[pl.ds(base, K)], idx_d)
        cps = (pltpu.async_copy(a_hbm.at[idx_s], r_as, sem),
               pltpu.async_copy(b_hbm.at[idx_d], r_bd, sem),
               pltpu.async_copy(a_hbm.at[idx_d], r_ad, sem),
               pltpu.async_copy(b_hbm.at[idx_s], r_bs, sem),
               pltpu.async_copy(c_hbm.at[pl.ds(base, K)], r_c, sem))
        for cp in cps:
            cp.wait()

        def _row(i, _):
            for j in range(D // 16):
                sl = pl.ds(j * 16, 16)
                c = r_c[i, sl]
                x1 = r_as[i, sl] + r_bd[i, sl] + c
                x2 = r_ad[i, sl] + r_bs[i, sl] + c
                r_as[i, sl] = jnp.maximum(x1, 0.01 * x1)
                r_ad[i, sl] = jnp.maximum(x2, 0.01 * x2)
            return _
        lax.fori_loop(0, K, _row, None)

        pltpu.sync_copy(r_as, s_sh.at[idx_d], add=True)
        pltpu.sync_copy(r_ad, s_sh.at[idx_s], add=True)
        return _
    lax.fori_loop(0, NCHUNK, _chunk, None)

    plsc.subcore_barrier()
    pltpu.sync_copy(s_sh.at[pl.ds(sid * ROWS_PER_SUB, ROWS_PER_SUB)],
                    out_hbm.at[cid, pl.ds(sid * ROWS_PER_SUB, ROWS_PER_SUB)])


@functools.lru_cache(maxsize=1)
def _edge_call():
    return pl.kernel(
        _edge_body,
        out_type=jax.ShapeDtypeStruct((NC, SPAD, D), jnp.float32),
        mesh=plsc.VectorSubcoreMesh(core_axis_name="c", subcore_axis_name="s"),
        scratch_types=[
        pltpu.VMEM_SHARED((SPAD, D), jnp.float32),
        pltpu.VMEM((K,), jnp.int32),
        pltpu.VMEM((K,), jnp.int32),
        pltpu.VMEM((K, D), jnp.float32),
        pltpu.VMEM((K, D), jnp.float32),
        pltpu.VMEM((K, D), jnp.float32),
        pltpu.VMEM((K, D), jnp.float32),
        pltpu.VMEM((K, D), jnp.float32),
        pltpu.SemaphoreType.DMA,
        ],
    )


def _prep_ab_body(nf_ref, wa_ref, wb_ref, a_ref, b_ref):
    x = nf_ref[...]
    a_ref[...] = x @ wa_ref[...]
    b_ref[...] = x @ wb_ref[...]


def _prep_c_body(ef_ref, wc_ref, c_ref):
    c_ref[...] = ef_ref[...] @ wc_ref[...]


def _node_body(sp_ref, nf_ref, we2_ref, wn1a_ref, wn1b_ref, wn2_ref, out_ref):
    red = (sp_ref[0] + sp_ref[1]) @ we2_ref[...]
    x = nf_ref[...] @ wn1a_ref[...] + red @ wn1b_ref[...]
    h = jnp.maximum(x, 0.01 * x)
    out_ref[...] = h @ wn2_ref[...]


_ROW_BLK = 400
_C_BLK = 2000


def kernel(nf, edge_index, ef, W_e1, W_e2, W_n1, W_n2):
    src = edge_index[0]
    dst = edge_index[1]

    A, B = pl.pallas_call(
        _prep_ab_body,
        grid=(N // _ROW_BLK,),
        in_specs=[
            pl.BlockSpec((_ROW_BLK, D), lambda i: (i, 0)),
            pl.BlockSpec((D, D), lambda i: (0, 0)),
            pl.BlockSpec((D, D), lambda i: (0, 0)),
        ],
        out_specs=[
            pl.BlockSpec((_ROW_BLK, D), lambda i: (i, 0)),
            pl.BlockSpec((_ROW_BLK, D), lambda i: (i, 0)),
        ],
        out_shape=[
            jax.ShapeDtypeStruct((N, D), jnp.float32),
            jax.ShapeDtypeStruct((N, D), jnp.float32),
        ],
    )(nf, W_e1[:D], W_e1[D:2 * D])

    C = pl.pallas_call(
        _prep_c_body,
        grid=(E // _C_BLK,),
        in_specs=[
            pl.BlockSpec((_C_BLK, 16), lambda i: (i, 0)),
            pl.BlockSpec((16, D), lambda i: (0, 0)),
        ],
        out_specs=pl.BlockSpec((_C_BLK, D), lambda i: (i, 0)),
        out_shape=jax.ShapeDtypeStruct((E, D), jnp.float32),
    )(ef, W_e1[2 * D:])

    s_parts = _edge_call()(A, B, C, src, dst)

    return pl.pallas_call(
        _node_body,
        grid=(N // _ROW_BLK,),
        in_specs=[
            pl.BlockSpec((NC, _ROW_BLK, D), lambda i: (0, i, 0)),
            pl.BlockSpec((_ROW_BLK, D), lambda i: (i, 0)),
            pl.BlockSpec((D, D), lambda i: (0, 0)),
            pl.BlockSpec((D, D), lambda i: (0, 0)),
            pl.BlockSpec((D, D), lambda i: (0, 0)),
            pl.BlockSpec((D, D), lambda i: (0, 0)),
        ],
        out_specs=pl.BlockSpec((_ROW_BLK, D), lambda i: (i, 0)),
        out_shape=jax.ShapeDtypeStruct((N, D), jnp.float32),
    )(s_parts, nf, W_e2, W_n1[:D], W_n1[D:], W_n2)


# speedup vs baseline: 8.5083x; 1.4958x over previous
import functools

import jax
import jax.numpy as jnp
from jax import lax
from jax.experimental import pallas as pl
from jax.experimental.pallas import tpu as pltpu
from jax.experimental.pallas import tpu_sc as plsc

N = 10000
E = 320000
D = 128

NC = 2
NS = 16
NW = NC * NS

SPAD = 10240
ROWS_PER_SUB = SPAD // NS
EDGES_PER_SUB = E // NW
K = 16
NCHUNK = EDGES_PER_SUB // K


def _edge_body(a_hbm, b_hbm, c_hbm, src_hbm, dst_hbm, out_hbm,
               s_sh, ixs_all, ixd_all, is0, is1, id0, id1,
               r_as, r_bd, r_ad, r_bs, r_c,
               semg0, semg1, semsc0, semsc1):
    cid = lax.axis_index("c")
    sid = lax.axis_index("s")
    wid = cid * NS + sid
    semg = (semg0, semg1)
    semsc = (semsc0, semsc1)
    isml = (is0, is1)
    idml = (id0, id1)

    def _zrow(i, _):
        for j in range(D // 16):
            r_c[0, i, pl.ds(j * 16, 16)] = jnp.zeros((16,), jnp.float32)
        return _
    lax.fori_loop(0, K, _zrow, None)

    def _zchunk(t, _):
        pltpu.sync_copy(r_c.at[0], s_sh.at[pl.ds(sid * ROWS_PER_SUB + t * K, K)])
        return _
    lax.fori_loop(0, ROWS_PER_SUB // K, _zchunk, None)

    ebase = wid * EDGES_PER_SUB
    pltpu.sync_copy(src_hbm.at[pl.ds(ebase, EDGES_PER_SUB)], ixs_all)
    pltpu.sync_copy(dst_hbm.at[pl.ds(ebase, EDGES_PER_SUB)], ixd_all)
    plsc.subcore_barrier()

    def _fill_idx(u, b):
        isml[b][...] = ixs_all[pl.ds(u * K, K)]
        idml[b][...] = ixd_all[pl.ds(u * K, K)]

    def _issue(u, b):
        pltpu.async_copy(a_hbm.at[isml[b]], r_as.at[b], semg[b])
        pltpu.async_copy(b_hbm.at[idml[b]], r_bd.at[b], semg[b])
        pltpu.async_copy(a_hbm.at[idml[b]], r_ad.at[b], semg[b])
        pltpu.async_copy(b_hbm.at[isml[b]], r_bs.at[b], semg[b])
        pltpu.async_copy(c_hbm.at[pl.ds(ebase + u * K, K)], r_c.at[b], semg[b])

    def _drain_g(b):
        for dst in (r_as, r_bd, r_ad, r_bs, r_c):
            pltpu.make_async_copy(c_hbm.at[pl.ds(0, K)], dst.at[b], semg[b]).wait()

    def _compute(b):
        def _row(i, _):
            for j in range(D // 16):
                sl = pl.ds(j * 16, 16)
                c = r_c[b, i, sl]
                x1 = r_as[b, i, sl] + r_bd[b, i, sl] + c
                x2 = r_ad[b, i, sl] + r_bs[b, i, sl] + c
                r_as[b, i, sl] = jnp.maximum(x1, 0.01 * x1)
                r_ad[b, i, sl] = jnp.maximum(x2, 0.01 * x2)
            return _
        lax.fori_loop(0, K, _row, None)

    def _issue_sc(b):
        pltpu.async_copy(r_as.at[b], s_sh.at[idml[b]], semsc[b], add=True)
        pltpu.async_copy(r_ad.at[b], s_sh.at[isml[b]], semsc[b], add=True)

    def _drain_sc(b):
        pltpu.make_async_copy(r_as.at[b], s_sh.at[idml[b]], semsc[b]).wait()
        pltpu.make_async_copy(r_ad.at[b], s_sh.at[isml[b]], semsc[b]).wait()

    _fill_idx(0, 0)
    _issue(0, 0)

    @pl.loop(0, NCHUNK, step=2)
    def _ring(t):
        for b in range(2):
            u = t + b
            nb = (b + 1) % 2

            @pl.when(u < NCHUNK)
            def _():
                @pl.when(u + 1 < NCHUNK)
                def _():
                    @pl.when(u >= 1)
                    def _():
                        _drain_sc(nb)
                    _fill_idx(u + 1, nb)
                    _issue(u + 1, nb)
                _drain_g(b)
                _compute(b)
                _issue_sc(b)

    _drain_sc(0)
    _drain_sc(1)

    plsc.subcore_barrier()
    pltpu.sync_copy(s_sh.at[pl.ds(sid * ROWS_PER_SUB, ROWS_PER_SUB)],
                    out_hbm.at[cid, pl.ds(sid * ROWS_PER_SUB, ROWS_PER_SUB)])


@functools.lru_cache(maxsize=1)
def _edge_call():
    return pl.kernel(
        _edge_body,
        out_type=jax.ShapeDtypeStruct((NC, SPAD, D), jnp.float32),
        mesh=plsc.VectorSubcoreMesh(core_axis_name="c", subcore_axis_name="s"),
        scratch_types=[
        pltpu.VMEM_SHARED((SPAD, D), jnp.float32),
        pltpu.VMEM((EDGES_PER_SUB,), jnp.int32),
        pltpu.VMEM((EDGES_PER_SUB,), jnp.int32),
        pltpu.VMEM((K,), jnp.int32),
        pltpu.VMEM((K,), jnp.int32),
        pltpu.VMEM((K,), jnp.int32),
        pltpu.VMEM((K,), jnp.int32),
        pltpu.VMEM((2, K, D), jnp.float32),
        pltpu.VMEM((2, K, D), jnp.float32),
        pltpu.VMEM((2, K, D), jnp.float32),
        pltpu.VMEM((2, K, D), jnp.float32),
        pltpu.VMEM((2, K, D), jnp.float32),
        pltpu.SemaphoreType.DMA,
        pltpu.SemaphoreType.DMA,
        pltpu.SemaphoreType.DMA,
        pltpu.SemaphoreType.DMA,
        ],
    )


def _prep_ab_body(nf_ref, wa_ref, wb_ref, a_ref, b_ref):
    x = nf_ref[...]
    a_ref[...] = x @ wa_ref[...]
    b_ref[...] = x @ wb_ref[...]


def _prep_c_body(ef_ref, wc_ref, c_ref):
    c_ref[...] = ef_ref[...] @ wc_ref[...]


def _node_body(sp_ref, nf_ref, we2_ref, wn1a_ref, wn1b_ref, wn2_ref, out_ref):
    red = (sp_ref[0] + sp_ref[1]) @ we2_ref[...]
    x = nf_ref[...] @ wn1a_ref[...] + red @ wn1b_ref[...]
    h = jnp.maximum(x, 0.01 * x)
    out_ref[...] = h @ wn2_ref[...]


_ROW_BLK = 400
_C_BLK = 2000


def kernel(nf, edge_index, ef, W_e1, W_e2, W_n1, W_n2):
    src = edge_index[0]
    dst = edge_index[1]

    A, B = pl.pallas_call(
        _prep_ab_body,
        grid=(N // _ROW_BLK,),
        in_specs=[
            pl.BlockSpec((_ROW_BLK, D), lambda i: (i, 0)),
            pl.BlockSpec((D, D), lambda i: (0, 0)),
            pl.BlockSpec((D, D), lambda i: (0, 0)),
        ],
        out_specs=[
            pl.BlockSpec((_ROW_BLK, D), lambda i: (i, 0)),
            pl.BlockSpec((_ROW_BLK, D), lambda i: (i, 0)),
        ],
        out_shape=[
            jax.ShapeDtypeStruct((N, D), jnp.float32),
            jax.ShapeDtypeStruct((N, D), jnp.float32),
        ],
    )(nf, W_e1[:D], W_e1[D:2 * D])

    C = pl.pallas_call(
        _prep_c_body,
        grid=(E // _C_BLK,),
        in_specs=[
            pl.BlockSpec((_C_BLK, 16), lambda i: (i, 0)),
            pl.BlockSpec((16, D), lambda i: (0, 0)),
        ],
        out_specs=pl.BlockSpec((_C_BLK, D), lambda i: (i, 0)),
        out_shape=jax.ShapeDtypeStruct((E, D), jnp.float32),
    )(ef, W_e1[2 * D:])

    s_parts = _edge_call()(A, B, C, src, dst)

    return pl.pallas_call(
        _node_body,
        grid=(N // _ROW_BLK,),
        in_specs=[
            pl.BlockSpec((NC, _ROW_BLK, D), lambda i: (0, i, 0)),
            pl.BlockSpec((_ROW_BLK, D), lambda i: (i, 0)),
            pl.BlockSpec((D, D), lambda i: (0, 0)),
            pl.BlockSpec((D, D), lambda i: (0, 0)),
            pl.BlockSpec((D, D), lambda i: (0, 0)),
            pl.BlockSpec((D, D), lambda i: (0, 0)),
        ],
        out_specs=pl.BlockSpec((_ROW_BLK, D), lambda i: (i, 0)),
        out_shape=jax.ShapeDtypeStruct((N, D), jnp.float32),
    )(s_parts, nf, W_e2, W_n1[:D], W_n1[D:], W_n2)
